# baseline (speedup 1.0000x reference)
import functools

import jax
import jax.numpy as jnp
import numpy as np
from jax import lax
from jax.experimental import pallas as pl
from jax.experimental.pallas import tpu as pltpu
from jax.experimental.pallas import tpu_sc as plsc

_BN = 1.0 / np.sqrt(1.0 + 1e-5)
_SUB = 8
_NW = 32


def _gather_rows_sc_body(table_hbm, gidx_hbm, out_hbm, idx_v, rows_v, sem,
                         *, R, W, CH):
    wid = lax.axis_index("s") * 2 + lax.axis_index("c")
    rpw = R // _NW
    base = wid * rpw

    def blk(i, _):
        off = base + i * CH
        pltpu.sync_copy(gidx_hbm.at[pl.ds(off, CH)], idx_v)
        pltpu.async_copy(table_hbm.at[idx_v], rows_v, sem).wait()
        pltpu.sync_copy(rows_v, out_hbm.at[pl.ds(off, CH)])
        return 0

    lax.fori_loop(0, rpw // CH, blk, 0)


def _gather_rows_sc(table, gidx):
    T, W = table.shape
    R = gidx.shape[0]
    CH = R // _NW
    while CH > 512 or (CH * W * 4 > 128 * 1024):
        CH //= 2
    while (R // _NW) % CH != 0:
        CH //= 2
    mesh = plsc.VectorSubcoreMesh(core_axis_name="c", subcore_axis_name="s")
    kfn = pl.kernel(
        functools.partial(_gather_rows_sc_body, R=R, W=W, CH=CH),
        out_type=jax.ShapeDtypeStruct((R, W), jnp.float32),
        mesh=mesh,
        compiler_params=pltpu.CompilerParams(use_tc_tiling_on_sc=False),
        scratch_types=[
            pltpu.VMEM((CH,), jnp.int32),
            pltpu.VMEM((CH, W), jnp.float32),
            pltpu.SemaphoreType.DMA,
        ],
    )
    return kfn(table, gidx)


def _fps_kbody(x_ref, y_ref, z_ref, xyzr_ref, inds_ref, cen_ref, *, npoint, N):
    LN = N // _SUB
    iota = (lax.broadcasted_iota(jnp.int32, (_SUB, LN), 0) * LN
            + lax.broadcasted_iota(jnp.int32, (_SUB, LN), 1))
    x = x_ref[...]
    y = y_ref[...]
    z = z_ref[...]

    def body(i, carry):
        dists, far = carry
        inds_ref[pl.ds(i, 1), :] = far[None, None]
        crow = xyzr_ref[pl.ds(far, 1), :]
        cen_ref[pl.ds(i, 1), :] = crow
        cx = crow[0, 0]
        cy = crow[0, 1]
        cz = crow[0, 2]
        dx = x - cx
        dy = y - cy
        dz = z - cz
        d = dx * dx + dy * dy
        d = d + dz * dz
        dists = jnp.minimum(dists, d)
        m = jnp.max(dists)
        far2 = jnp.min(jnp.where(dists == m, iota, N)).astype(jnp.int32)
        return dists, far2

    dists0 = jnp.full((_SUB, LN), 1e10, jnp.float32)
    lax.fori_loop(0, npoint, body, (dists0, jnp.int32(0)))


def _fps_pallas(xyz, npoint):
    B, N, _ = xyz.shape
    LN = N // _SUB
    xt = jnp.moveaxis(xyz, -1, 1).reshape(B, 3, _SUB, LN)
    x, y, z = xt[:, 0], xt[:, 1], xt[:, 2]
    inds, cen = pl.pallas_call(
        functools.partial(_fps_kbody, npoint=npoint, N=N),
        grid=(B,),
        in_specs=[
            pl.BlockSpec((None, _SUB, LN), lambda b: (b, 0, 0)),
            pl.BlockSpec((None, _SUB, LN), lambda b: (b, 0, 0)),
            pl.BlockSpec((None, _SUB, LN), lambda b: (b, 0, 0)),
            pl.BlockSpec((None, N, 3), lambda b: (b, 0, 0)),
        ],
        out_specs=[
            pl.BlockSpec((None, npoint, 1), lambda b: (b, 0, 0)),
            pl.BlockSpec((None, npoint, 3), lambda b: (b, 0, 0)),
        ],
        out_shape=[
            jax.ShapeDtypeStruct((B, npoint, 1), jnp.int32),
            jax.ShapeDtypeStruct((B, npoint, 3), jnp.float32),
        ],
    )(x, y, z, xyz)
    return inds[..., 0], cen


def _fps2_kbody(x_ref, y_ref, z_ref, xyzr_ref, inds_ref, cen_ref, *, npoint, N, B):
    LN = N // _SUB
    iota = (lax.broadcasted_iota(jnp.int32, (_SUB, LN), 0) * LN
            + lax.broadcasted_iota(jnp.int32, (_SUB, LN), 1))
    xs = [x_ref[b] for b in range(B)]
    ys = [y_ref[b] for b in range(B)]
    zs = [z_ref[b] for b in range(B)]

    def body(i, carry):
        dists, far = carry
        dists2, far2 = [], []
        for b in range(B):
            inds_ref[b, pl.ds(i, 1), :] = far[b][None, None]
            crow = xyzr_ref[b, pl.ds(far[b], 1), :]
            cen_ref[b, pl.ds(i, 1), :] = crow
            cx = crow[0, 0]
            cy = crow[0, 1]
            cz = crow[0, 2]
            dx = xs[b] - cx
            dy = ys[b] - cy
            dz = zs[b] - cz
            d = dx * dx + dy * dy
            d = d + dz * dz
            db = jnp.minimum(dists[b], d)
            m = jnp.max(db)
            fb = jnp.min(jnp.where(db == m, iota, N)).astype(jnp.int32)
            dists2.append(db)
            far2.append(fb)
        return tuple(dists2), tuple(far2)

    d0 = tuple(jnp.full((_SUB, LN), 1e10, jnp.float32) for _ in range(B))
    f0 = tuple(jnp.int32(0) for _ in range(B))
    lax.fori_loop(0, npoint, body, (d0, f0))


def fps2_pallas(xyz, npoint):
    B, N, _ = xyz.shape
    LN = N // _SUB
    xt = jnp.moveaxis(xyz, -1, 1).reshape(B, 3, _SUB, LN)
    x, y, z = xt[:, 0], xt[:, 1], xt[:, 2]
    inds, cen = pl.pallas_call(
        functools.partial(_fps2_kbody, npoint=npoint, N=N, B=B),
        grid=(),
        in_specs=[
            pl.BlockSpec((B, _SUB, LN), lambda: (0, 0, 0)),
            pl.BlockSpec((B, _SUB, LN), lambda: (0, 0, 0)),
            pl.BlockSpec((B, _SUB, LN), lambda: (0, 0, 0)),
            pl.BlockSpec((B, N, 3), lambda: (0, 0, 0)),
        ],
        out_specs=[
            pl.BlockSpec((B, npoint, 1), lambda: (0, 0, 0)),
            pl.BlockSpec((B, npoint, 3), lambda: (0, 0, 0)),
        ],
        out_shape=[
            jax.ShapeDtypeStruct((B, npoint, 1), jnp.int32),
            jax.ShapeDtypeStruct((B, npoint, 3), jnp.float32),
        ],
    )(x, y, z, xyz)
    return inds[..., 0], cen


def _fold(t, op):
    w = t.shape[1]
    while w > 1:
        w //= 2
        t = op(t[:, :w], t[:, w:2 * w])
    s = t.shape[0]
    while s > 1:
        s //= 2
        t = op(t[:s], t[s:2 * s])
    return t


def _fps3_kbody(x_ref, y_ref, z_ref, inds_ref, cen_ref, *, npoint, N, B):
    LN = N // _SUB
    iota = (lax.broadcasted_iota(jnp.int32, (_SUB, LN), 0) * LN
            + lax.broadcasted_iota(jnp.int32, (_SUB, LN), 1))
    xs = [x_ref[b] for b in range(B)]
    ys = [y_ref[b] for b in range(B)]
    zs = [z_ref[b] for b in range(B)]

    def body(i, carry):
        dists, fb = carry
        dists2, fb2 = [], []
        for b in range(B):
            inds_ref[b, pl.ds(i, 1), :] = fb[b]
            onehot = iota == fb[b]
            zero = jnp.zeros((_SUB, LN), jnp.float32)
            cx = _fold(jnp.where(onehot, xs[b], zero), jnp.add)
            cy = _fold(jnp.where(onehot, ys[b], zero), jnp.add)
            cz = _fold(jnp.where(onehot, zs[b], zero), jnp.add)
            cen_ref[b, pl.ds(i, 1), :] = jnp.concatenate([cx, cy, cz], axis=1)
            dx = xs[b] - cx
            dy = ys[b] - cy
            dz = zs[b] - cz
            d = dx * dx + dy * dy
            d = d + dz * dz
            db = jnp.minimum(dists[b], d)
            m = _fold(db, jnp.maximum)
            f = _fold(jnp.where(db == m, iota, N), jnp.minimum)
            dists2.append(db)
            fb2.append(f)
        return tuple(dists2), tuple(fb2)

    d0 = tuple(jnp.full((_SUB, LN), 1e10, jnp.float32) for _ in range(B))
    f0 = tuple(jnp.zeros((1, 1), jnp.int32) for _ in range(B))
    lax.fori_loop(0, npoint, body, (d0, f0))


def fps3_pallas(xyz, npoint):
    B, N, _ = xyz.shape
    LN = N // _SUB
    xt = jnp.moveaxis(xyz, -1, 1).reshape(B, 3, _SUB, LN)
    x, y, z = xt[:, 0], xt[:, 1], xt[:, 2]
    inds, cen = pl.pallas_call(
        functools.partial(_fps3_kbody, npoint=npoint, N=N, B=B),
        grid=(),
        in_specs=[
            pl.BlockSpec((B, _SUB, LN), lambda: (0, 0, 0)),
            pl.BlockSpec((B, _SUB, LN), lambda: (0, 0, 0)),
            pl.BlockSpec((B, _SUB, LN), lambda: (0, 0, 0)),
        ],
        out_specs=[
            pl.BlockSpec((B, npoint, 1), lambda: (0, 0, 0)),
            pl.BlockSpec((B, npoint, 3), lambda: (0, 0, 0)),
        ],
        out_shape=[
            jax.ShapeDtypeStruct((B, npoint, 1), jnp.int32),
            jax.ShapeDtypeStruct((B, npoint, 3), jnp.float32),
        ],
    )(x, y, z)
    return inds[..., 0], cen


def _mlp_max_kbody(rows_ref, cen_ref, w1_ref, b1_ref, w2_ref, b2_ref,
                   w3_ref, b3_ref, out_ref, *, K, radius):
    rows = rows_ref[...]
    TR = rows.shape[0]
    G = TR // K
    c3 = cen_ref[...][:, 0:3]
    cb = jnp.broadcast_to(c3[:, None, :], (G, K, 3)).reshape(TR, 3)
    gxn = (rows[:, 0:3] - cb) / radius
    x_in = jnp.concatenate([gxn, rows[:, 3:]], axis=1)
    h = jnp.dot(x_in, w1_ref[...], preferred_element_type=jnp.float32)
    h = jnp.maximum((h + b1_ref[...]) * _BN, 0.0)
    h = jnp.dot(h, w2_ref[...], preferred_element_type=jnp.float32)
    h = jnp.maximum((h + b2_ref[...]) * _BN, 0.0)
    h = jnp.dot(h, w3_ref[...], preferred_element_type=jnp.float32)
    h = jnp.maximum((h + b3_ref[...]) * _BN, 0.0)
    out_ref[...] = jnp.max(h.reshape(G, K, h.shape[1]), axis=1)


def _mlp_max_pallas(rows, cen, layers, K, radius):
    R, Wpad = rows.shape
    (W1, b1), (W2, b2), (W3, b3) = layers
    W1p = jnp.concatenate(
        [W1, jnp.zeros((Wpad - W1.shape[0], W1.shape[1]), jnp.float32)], axis=0)
    TR = 2048
    while R % TR != 0:
        TR //= 2
    grid = (R // TR,)
    Cout = W3.shape[1]
    zer = lambda i: (0, 0)
    out = pl.pallas_call(
        functools.partial(_mlp_max_kbody, K=K, radius=radius),
        grid=grid,
        in_specs=[
            pl.BlockSpec((TR, Wpad), lambda i: (i, 0)),
            pl.BlockSpec((TR // K, 8), lambda i: (i, 0)),
            pl.BlockSpec(W1p.shape, zer), pl.BlockSpec((1, b1.shape[0]), zer),
            pl.BlockSpec(W2.shape, zer), pl.BlockSpec((1, b2.shape[0]), zer),
            pl.BlockSpec(W3.shape, zer), pl.BlockSpec((1, b3.shape[0]), zer),
        ],
        out_specs=pl.BlockSpec((TR // K, Cout), lambda i: (i, 0)),
        out_shape=jax.ShapeDtypeStruct((R // K, Cout), jnp.float32),
    )(rows, cen, W1p, b1[None], W2, b2[None], W3, b3[None])
    return out


def _bqidx_kbody(pts_ref, cen_ref, idx_ref, *, N, K, CH, r2):
    TS = cen_ref.shape[0]
    xn = pts_ref[0:1, :]
    yn = pts_ref[1:2, :]
    zn = pts_ref[2:3, :]
    cx = cen_ref[:, 0:1]
    cy = cen_ref[:, 1:2]
    cz = cen_ref[:, 2:3]
    dx = cx - xn
    dy = cy - yn
    dz = cz - zn
    d2 = dx * dx + dy * dy
    d2 = d2 + dz * dz
    mf = jnp.where(d2 <= r2, 1.0, 0.0).astype(jnp.float32)
    l128a = lax.broadcasted_iota(jnp.int32, (128, 128), 0)
    l128b = lax.broadcasted_iota(jnp.int32, (128, 128), 1)
    T128 = jnp.where(l128a <= l128b, 1.0, 0.0).astype(jnp.float32)
    m3 = mf.reshape(TS, CH, 128)
    ic3 = lax.dot_general(m3, T128, (((2,), (0,)), ((), ())),
                          preferred_element_type=jnp.float32)
    cnt = ic3[:, :, 127]
    lca = lax.broadcasted_iota(jnp.int32, (CH, CH), 0)
    lcb = lax.broadcasted_iota(jnp.int32, (CH, CH), 1)
    TCH = jnp.where(lca <= lcb, 1.0, 0.0).astype(jnp.float32)
    Cin = jnp.dot(cnt, TCH, preferred_element_type=jnp.float32)
    Cex = Cin - cnt
    kio = lax.broadcasted_iota(jnp.int32, (1, 1, K), 2).astype(jnp.float32)
    below_in = jnp.where(Cin[:, :, None] <= kio, 1.0, 0.0)
    below_ex = jnp.where(Cex[:, :, None] <= kio, 1.0, 0.0)
    F = jnp.sum(below_in, axis=1)
    P = below_ex - below_in
    Cex_sel = jnp.sum(P * Cex[:, :, None], axis=1)
    hasP = jnp.sum(P, axis=1)
    sel = lax.dot_general(P, ic3, (((1,), (1,)), ((0,), (0,))),
                          preferred_element_type=jnp.float32)
    kio2 = lax.broadcasted_iota(jnp.int32, (TS, K), 1).astype(jnp.float32)
    t = kio2 - Cex_sel
    partial = jnp.sum(jnp.where(sel <= t[:, :, None], 1.0, 0.0), axis=2) * hasP
    idxi = (128.0 * F + partial).astype(jnp.int32)
    first = idxi[:, 0:1]
    idx_ref[...] = jnp.where(idxi >= N, jnp.broadcast_to(first, (TS, K)), idxi)


def bq_idx_pallas(xyz, cen, radius, K, interpret=False):
    B, N, _ = xyz.shape
    S = cen.shape[1]
    CH = N // 128
    r2 = np.float32(radius * radius)
    TS = min(128, S)
    cen_p = jnp.concatenate(
        [cen, jnp.zeros((B, S, 5), jnp.float32)], axis=-1)
    pts = xyz.transpose(0, 2, 1)
    idx = pl.pallas_call(
        functools.partial(_bqidx_kbody, N=N, K=K, CH=CH, r2=r2),
        grid=(B, S // TS),
        in_specs=[
            pl.BlockSpec((None, 3, N), lambda b, s: (b, 0, 0)),
            pl.BlockSpec((None, TS, 8), lambda b, s: (b, s, 0)),
        ],
        out_specs=pl.BlockSpec((None, TS, K), lambda b, s: (b, s, 0)),
        out_shape=jax.ShapeDtypeStruct((B, S, K), jnp.int32),
        interpret=interpret,
    )(pts, cen_p)
    return idx


def _mlp(x, layers):
    for (W, b) in layers:
        x = jax.nn.relu((x @ W + b) * _BN)
    return x


def _gath(x, idx):
    if idx.ndim == 2:
        return jnp.take_along_axis(x, idx[..., None], axis=1)
    B, S, K = idx.shape
    g = jnp.take_along_axis(x, idx.reshape(B, S * K)[..., None], axis=1)
    return g.reshape(B, S, K, x.shape[-1])


def _bq(radius, nsample, xyz, new_xyz):
    d2 = jnp.sum((new_xyz[:, :, None, :] - xyz[:, None, :, :]) ** 2, -1)
    N = xyz.shape[1]
    order = jnp.where(d2 <= radius * radius,
                      jnp.arange(N, dtype=jnp.float32)[None, None, :], jnp.inf)
    _, idx = jax.lax.top_k(-order, nsample)
    valid = jnp.take_along_axis(order, idx, axis=-1) < jnp.inf
    idx = jnp.where(valid, idx, idx[:, :, :1])
    return idx


def _sa_x(xyz, feats, npoint, radius, nsample, layers):
    B, N, _ = xyz.shape
    C = feats.shape[-1]
    K = nsample
    inds, new_xyz = fps3_pallas(xyz, npoint)
    idx = bq_idx_pallas(xyz, new_xyz, radius, K)
    Wpad = ((3 + C + 7) // 8) * 8
    table = jnp.concatenate(
        [xyz, feats, jnp.zeros((B, N, Wpad - 3 - C), jnp.float32)],
        axis=-1).reshape(B * N, Wpad)
    gidx = (idx + (jnp.arange(B, dtype=jnp.int32) * N)[:, None, None]).reshape(-1)
    rows = _gather_rows_sc(table, gidx)
    cen = jnp.concatenate(
        [new_xyz, jnp.zeros((B, npoint, 5), jnp.float32)], axis=-1
    ).reshape(B * npoint, 8)
    pooled = _mlp_max_pallas(rows, cen, layers, K, radius)
    return new_xyz, pooled.reshape(B, npoint, -1), inds


def _fp_kbody(u_ref, kT_ref, uf_ref, kf_ref, w1_ref, b1_ref, w2_ref, b2_ref,
              out_ref, *, NU, NK):
    ux = u_ref[...][:, 0:1]
    uy = u_ref[...][:, 1:2]
    uz = u_ref[...][:, 2:3]
    kx = kT_ref[0:1, :]
    ky = kT_ref[1:2, :]
    kz = kT_ref[2:3, :]
    dx = ux - kx
    dy = uy - ky
    dz = uz - kz
    d2 = dx * dx + dy * dy
    d2 = d2 + dz * dz
    iota_k = lax.broadcasted_iota(jnp.int32, (NU, NK), 1)
    A = jnp.zeros((NU, NK), jnp.float32)
    rsum = jnp.zeros((NU, 1), jnp.float32)
    for _ in range(3):
        mj = jnp.min(d2, axis=1, keepdims=True)
        idxj = jnp.min(jnp.where(d2 == mj, iota_k, NK), axis=1, keepdims=True)
        onehot = jnp.where(iota_k == idxj, 1.0, 0.0)
        recip = 1.0 / (mj + 1e-8)
        A = A + recip * onehot
        rsum = rsum + recip
        d2 = jnp.where(iota_k == idxj, jnp.inf, d2)
    A = A / rsum
    interp = jnp.dot(A, kf_ref[...], preferred_element_type=jnp.float32)
    x = jnp.concatenate([interp, uf_ref[...]], axis=1)
    h = jnp.dot(x, w1_ref[...], preferred_element_type=jnp.float32)
    h = jnp.maximum((h + b1_ref[...]) * _BN, 0.0)
    h = jnp.dot(h, w2_ref[...], preferred_element_type=jnp.float32)
    h = jnp.maximum((h + b2_ref[...]) * _BN, 0.0)
    out_ref[...] = h


def fp_pallas(unknown, known, unknown_feats, known_feats, layers, interpret=False):
    B, NU, _ = unknown.shape
    NK = known.shape[1]
    CU = unknown_feats.shape[-1]
    CK = known_feats.shape[-1]
    (W1, b1), (W2, b2) = layers
    Cout = W2.shape[1]
    kT = known.transpose(0, 2, 1)
    zer2 = lambda b: (0, 0)
    out = pl.pallas_call(
        functools.partial(_fp_kbody, NU=NU, NK=NK),
        grid=(B,),
        in_specs=[
            pl.BlockSpec((None, NU, 3), lambda b: (b, 0, 0)),
            pl.BlockSpec((None, 3, NK), lambda b: (b, 0, 0)),
            pl.BlockSpec((None, NU, CU), lambda b: (b, 0, 0)),
            pl.BlockSpec((None, NK, CK), lambda b: (b, 0, 0)),
            pl.BlockSpec(W1.shape, zer2), pl.BlockSpec((1, b1.shape[0]), zer2),
            pl.BlockSpec(W2.shape, zer2), pl.BlockSpec((1, b2.shape[0]), zer2),
        ],
        out_specs=pl.BlockSpec((None, NU, Cout), lambda b: (b, 0, 0)),
        out_shape=jax.ShapeDtypeStruct((B, NU, Cout), jnp.float32),
        interpret=interpret,
    )(unknown, kT, unknown_feats, known_feats, W1, b1[None], W2, b2[None])
    return out


def _fp_x(unknown, known, unknown_feats, known_feats, layers):
    return fp_pallas(unknown, known, unknown_feats, known_feats, layers)


def kernel(pointcloud, params):
    xyz = pointcloud[..., 0:3]
    feats = pointcloud[..., 3:]
    sa1_xyz, sa1_f, sa1_inds = _sa_x(xyz, feats, 2048, 0.2, 64, params['sa1'])
    sa2_xyz, sa2_f, sa2_inds = _sa_x(sa1_xyz, sa1_f, 1024, 0.4, 32, params['sa2'])
    sa3_xyz, sa3_f, sa3_inds = _sa_x(sa2_xyz, sa2_f, 512, 0.8, 16, params['sa3'])
    sa4_xyz, sa4_f, sa4_inds = _sa_x(sa3_xyz, sa3_f, 256, 1.2, 16, params['sa4'])
    f = _fp_x(sa3_xyz, sa4_xyz, sa3_f, sa4_f, params['fp1'])
    fp2_f = _fp_x(sa2_xyz, sa3_xyz, sa2_f, f, params['fp2'])
    fp2_inds = sa1_inds[:, :fp2_f.shape[1]]
    return (sa1_xyz, sa1_f, sa1_inds, sa2_xyz, sa2_f, sa2_inds, sa3_xyz, sa3_f,
            sa4_xyz, sa4_f, fp2_f, sa2_xyz, fp2_inds)


# speedup vs baseline: 2.8281x; 2.8281x over previous
import functools

import jax
import jax.numpy as jnp
import numpy as np
from jax import lax
from jax.experimental import pallas as pl
from jax.experimental.pallas import tpu as pltpu
from jax.experimental.pallas import tpu_sc as plsc

_BN = 1.0 / np.sqrt(1.0 + 1e-5)
_SUB = 8
_NW = 32


def _gather_rows_sc_body(table_hbm, gidx_hbm, out_hbm, idx_v, rows_v, sem,
                         *, R, W, CH):
    wid = lax.axis_index("s") * 2 + lax.axis_index("c")
    rpw = R // _NW
    base = wid * rpw

    def blk(i, _):
        off = base + i * CH
        pltpu.sync_copy(gidx_hbm.at[pl.ds(off, CH)], idx_v)
        pltpu.async_copy(table_hbm.at[idx_v], rows_v, sem).wait()
        pltpu.sync_copy(rows_v, out_hbm.at[pl.ds(off, CH)])
        return 0

    lax.fori_loop(0, rpw // CH, blk, 0)


def _gather_rows_sc(table, gidx):
    T, W = table.shape
    R = gidx.shape[0]
    CH = R // _NW
    while CH > 512 or (CH * W * 4 > 128 * 1024):
        CH //= 2
    while (R // _NW) % CH != 0:
        CH //= 2
    mesh = plsc.VectorSubcoreMesh(core_axis_name="c", subcore_axis_name="s")
    kfn = pl.kernel(
        functools.partial(_gather_rows_sc_body, R=R, W=W, CH=CH),
        out_type=jax.ShapeDtypeStruct((R, W), jnp.float32),
        mesh=mesh,
        compiler_params=pltpu.CompilerParams(use_tc_tiling_on_sc=False),
        scratch_types=[
            pltpu.VMEM((CH,), jnp.int32),
            pltpu.VMEM((CH, W), jnp.float32),
            pltpu.SemaphoreType.DMA,
        ],
    )
    return kfn(table, gidx)


def _fps_kbody(x_ref, y_ref, z_ref, xyzr_ref, inds_ref, cen_ref, *, npoint, N):
    LN = N // _SUB
    iota = (lax.broadcasted_iota(jnp.int32, (_SUB, LN), 0) * LN
            + lax.broadcasted_iota(jnp.int32, (_SUB, LN), 1))
    x = x_ref[...]
    y = y_ref[...]
    z = z_ref[...]

    def body(i, carry):
        dists, far = carry
        inds_ref[pl.ds(i, 1), :] = far[None, None]
        crow = xyzr_ref[pl.ds(far, 1), :]
        cen_ref[pl.ds(i, 1), :] = crow
        cx = crow[0, 0]
        cy = crow[0, 1]
        cz = crow[0, 2]
        dx = x - cx
        dy = y - cy
        dz = z - cz
        d = dx * dx + dy * dy
        d = d + dz * dz
        dists = jnp.minimum(dists, d)
        m = jnp.max(dists)
        far2 = jnp.min(jnp.where(dists == m, iota, N)).astype(jnp.int32)
        return dists, far2

    dists0 = jnp.full((_SUB, LN), 1e10, jnp.float32)
    lax.fori_loop(0, npoint, body, (dists0, jnp.int32(0)))


def _fps_pallas(xyz, npoint):
    B, N, _ = xyz.shape
    LN = N // _SUB
    xt = jnp.moveaxis(xyz, -1, 1).reshape(B, 3, _SUB, LN)
    x, y, z = xt[:, 0], xt[:, 1], xt[:, 2]
    inds, cen = pl.pallas_call(
        functools.partial(_fps_kbody, npoint=npoint, N=N),
        grid=(B,),
        in_specs=[
            pl.BlockSpec((None, _SUB, LN), lambda b: (b, 0, 0)),
            pl.BlockSpec((None, _SUB, LN), lambda b: (b, 0, 0)),
            pl.BlockSpec((None, _SUB, LN), lambda b: (b, 0, 0)),
            pl.BlockSpec((None, N, 3), lambda b: (b, 0, 0)),
        ],
        out_specs=[
            pl.BlockSpec((None, npoint, 1), lambda b: (b, 0, 0)),
            pl.BlockSpec((None, npoint, 3), lambda b: (b, 0, 0)),
        ],
        out_shape=[
            jax.ShapeDtypeStruct((B, npoint, 1), jnp.int32),
            jax.ShapeDtypeStruct((B, npoint, 3), jnp.float32),
        ],
    )(x, y, z, xyz)
    return inds[..., 0], cen


def _fps2_kbody(x_ref, y_ref, z_ref, xyzr_ref, inds_ref, cen_ref, *, npoint, N, B):
    LN = N // _SUB
    iota = (lax.broadcasted_iota(jnp.int32, (_SUB, LN), 0) * LN
            + lax.broadcasted_iota(jnp.int32, (_SUB, LN), 1))
    xs = [x_ref[b] for b in range(B)]
    ys = [y_ref[b] for b in range(B)]
    zs = [z_ref[b] for b in range(B)]

    def body(i, carry):
        dists, far = carry
        dists2, far2 = [], []
        for b in range(B):
            inds_ref[b, pl.ds(i, 1), :] = far[b][None, None]
            crow = xyzr_ref[b, pl.ds(far[b], 1), :]
            cen_ref[b, pl.ds(i, 1), :] = crow
            cx = crow[0, 0]
            cy = crow[0, 1]
            cz = crow[0, 2]
            dx = xs[b] - cx
            dy = ys[b] - cy
            dz = zs[b] - cz
            d = dx * dx + dy * dy
            d = d + dz * dz
            db = jnp.minimum(dists[b], d)
            m = jnp.max(db)
            fb = jnp.min(jnp.where(db == m, iota, N)).astype(jnp.int32)
            dists2.append(db)
            far2.append(fb)
        return tuple(dists2), tuple(far2)

    d0 = tuple(jnp.full((_SUB, LN), 1e10, jnp.float32) for _ in range(B))
    f0 = tuple(jnp.int32(0) for _ in range(B))
    lax.fori_loop(0, npoint, body, (d0, f0))


def fps2_pallas(xyz, npoint):
    B, N, _ = xyz.shape
    LN = N // _SUB
    xt = jnp.moveaxis(xyz, -1, 1).reshape(B, 3, _SUB, LN)
    x, y, z = xt[:, 0], xt[:, 1], xt[:, 2]
    inds, cen = pl.pallas_call(
        functools.partial(_fps2_kbody, npoint=npoint, N=N, B=B),
        grid=(),
        in_specs=[
            pl.BlockSpec((B, _SUB, LN), lambda: (0, 0, 0)),
            pl.BlockSpec((B, _SUB, LN), lambda: (0, 0, 0)),
            pl.BlockSpec((B, _SUB, LN), lambda: (0, 0, 0)),
            pl.BlockSpec((B, N, 3), lambda: (0, 0, 0)),
        ],
        out_specs=[
            pl.BlockSpec((B, npoint, 1), lambda: (0, 0, 0)),
            pl.BlockSpec((B, npoint, 3), lambda: (0, 0, 0)),
        ],
        out_shape=[
            jax.ShapeDtypeStruct((B, npoint, 1), jnp.int32),
            jax.ShapeDtypeStruct((B, npoint, 3), jnp.float32),
        ],
    )(x, y, z, xyz)
    return inds[..., 0], cen


def _fold(t, op):
    w = t.shape[1]
    while w > 1:
        w //= 2
        t = op(t[:, :w], t[:, w:2 * w])
    s = t.shape[0]
    while s > 1:
        s //= 2
        t = op(t[:s], t[s:2 * s])
    return t


def _fps3_kbody(x_ref, y_ref, z_ref, inds_ref, cen_ref, *, npoint, N, B):
    LN = N // _SUB
    iota = (lax.broadcasted_iota(jnp.int32, (_SUB, LN), 0) * LN
            + lax.broadcasted_iota(jnp.int32, (_SUB, LN), 1))
    xs = [x_ref[b] for b in range(B)]
    ys = [y_ref[b] for b in range(B)]
    zs = [z_ref[b] for b in range(B)]

    def body(i, carry):
        dists, fb = carry
        dists2, fb2 = [], []
        for b in range(B):
            inds_ref[b, pl.ds(i, 1), :] = fb[b]
            onehot = iota == fb[b]
            zero = jnp.zeros((_SUB, LN), jnp.float32)
            cx = _fold(jnp.where(onehot, xs[b], zero), jnp.add)
            cy = _fold(jnp.where(onehot, ys[b], zero), jnp.add)
            cz = _fold(jnp.where(onehot, zs[b], zero), jnp.add)
            cen_ref[b, pl.ds(i, 1), :] = jnp.concatenate([cx, cy, cz], axis=1)
            dx = xs[b] - cx
            dy = ys[b] - cy
            dz = zs[b] - cz
            d = dx * dx + dy * dy
            d = d + dz * dz
            db = jnp.minimum(dists[b], d)
            m = _fold(db, jnp.maximum)
            f = _fold(jnp.where(db == m, iota, N), jnp.minimum)
            dists2.append(db)
            fb2.append(f)
        return tuple(dists2), tuple(fb2)

    d0 = tuple(jnp.full((_SUB, LN), 1e10, jnp.float32) for _ in range(B))
    f0 = tuple(jnp.zeros((1, 1), jnp.int32) for _ in range(B))
    lax.fori_loop(0, npoint, body, (d0, f0))


def fps3_pallas(xyz, npoint):
    B, N, _ = xyz.shape
    LN = N // _SUB
    xt = jnp.moveaxis(xyz, -1, 1).reshape(B, 3, _SUB, LN)
    x, y, z = xt[:, 0], xt[:, 1], xt[:, 2]
    inds, cen = pl.pallas_call(
        functools.partial(_fps3_kbody, npoint=npoint, N=N, B=B),
        grid=(),
        in_specs=[
            pl.BlockSpec((B, _SUB, LN), lambda: (0, 0, 0)),
            pl.BlockSpec((B, _SUB, LN), lambda: (0, 0, 0)),
            pl.BlockSpec((B, _SUB, LN), lambda: (0, 0, 0)),
        ],
        out_specs=[
            pl.BlockSpec((B, npoint, 1), lambda: (0, 0, 0)),
            pl.BlockSpec((B, npoint, 3), lambda: (0, 0, 0)),
        ],
        out_shape=[
            jax.ShapeDtypeStruct((B, npoint, 1), jnp.int32),
            jax.ShapeDtypeStruct((B, npoint, 3), jnp.float32),
        ],
    )(x, y, z)
    return inds[..., 0], cen


def _mlp_max_kbody(rows_ref, cen_ref, w1_ref, b1_ref, w2_ref, b2_ref,
                   w3_ref, b3_ref, out_ref, *, K, radius):
    rows = rows_ref[...]
    TR = rows.shape[0]
    G = TR // K
    c3 = cen_ref[...][:, 0:3]
    cb = jnp.broadcast_to(c3[:, None, :], (G, K, 3)).reshape(TR, 3)
    gxn = (rows[:, 0:3] - cb) / radius
    x_in = jnp.concatenate([gxn, rows[:, 3:]], axis=1)
    h = jnp.dot(x_in, w1_ref[...], preferred_element_type=jnp.float32)
    h = jnp.maximum((h + b1_ref[...]) * _BN, 0.0)
    h = jnp.dot(h, w2_ref[...], preferred_element_type=jnp.float32)
    h = jnp.maximum((h + b2_ref[...]) * _BN, 0.0)
    h = jnp.dot(h, w3_ref[...], preferred_element_type=jnp.float32)
    h = jnp.maximum((h + b3_ref[...]) * _BN, 0.0)
    out_ref[...] = jnp.max(h.reshape(G, K, h.shape[1]), axis=1)


def _mlp_max_pallas(rows, cen, layers, K, radius):
    R, Wpad = rows.shape
    (W1, b1), (W2, b2), (W3, b3) = layers
    W1p = jnp.concatenate(
        [W1, jnp.zeros((Wpad - W1.shape[0], W1.shape[1]), jnp.float32)], axis=0)
    TR = 2048
    while R % TR != 0:
        TR //= 2
    grid = (R // TR,)
    Cout = W3.shape[1]
    zer = lambda i: (0, 0)
    out = pl.pallas_call(
        functools.partial(_mlp_max_kbody, K=K, radius=radius),
        grid=grid,
        in_specs=[
            pl.BlockSpec((TR, Wpad), lambda i: (i, 0)),
            pl.BlockSpec((TR // K, 8), lambda i: (i, 0)),
            pl.BlockSpec(W1p.shape, zer), pl.BlockSpec((1, b1.shape[0]), zer),
            pl.BlockSpec(W2.shape, zer), pl.BlockSpec((1, b2.shape[0]), zer),
            pl.BlockSpec(W3.shape, zer), pl.BlockSpec((1, b3.shape[0]), zer),
        ],
        out_specs=pl.BlockSpec((TR // K, Cout), lambda i: (i, 0)),
        out_shape=jax.ShapeDtypeStruct((R // K, Cout), jnp.float32),
    )(rows, cen, W1p, b1[None], W2, b2[None], W3, b3[None])
    return out


def _bqidx_kbody(pts_ref, cen_ref, idx_ref, *, N, K, CH, r2):
    TS = cen_ref.shape[0]
    xn = pts_ref[0:1, :]
    yn = pts_ref[1:2, :]
    zn = pts_ref[2:3, :]
    cx = cen_ref[:, 0:1]
    cy = cen_ref[:, 1:2]
    cz = cen_ref[:, 2:3]
    dx = cx - xn
    dy = cy - yn
    dz = cz - zn
    d2 = dx * dx + dy * dy
    d2 = d2 + dz * dz
    mf = jnp.where(d2 <= r2, 1.0, 0.0).astype(jnp.float32)
    l128a = lax.broadcasted_iota(jnp.int32, (128, 128), 0)
    l128b = lax.broadcasted_iota(jnp.int32, (128, 128), 1)
    T128 = jnp.where(l128a <= l128b, 1.0, 0.0).astype(jnp.float32)
    m3 = mf.reshape(TS, CH, 128)
    ic3 = lax.dot_general(m3, T128, (((2,), (0,)), ((), ())),
                          preferred_element_type=jnp.float32)
    cnt = ic3[:, :, 127]
    lca = lax.broadcasted_iota(jnp.int32, (CH, CH), 0)
    lcb = lax.broadcasted_iota(jnp.int32, (CH, CH), 1)
    TCH = jnp.where(lca <= lcb, 1.0, 0.0).astype(jnp.float32)
    Cin = jnp.dot(cnt, TCH, preferred_element_type=jnp.float32)
    Cex = Cin - cnt
    kio = lax.broadcasted_iota(jnp.int32, (1, 1, K), 2).astype(jnp.float32)
    below_in = jnp.where(Cin[:, :, None] <= kio, 1.0, 0.0)
    below_ex = jnp.where(Cex[:, :, None] <= kio, 1.0, 0.0)
    F = jnp.sum(below_in, axis=1)
    P = below_ex - below_in
    Cex_sel = jnp.sum(P * Cex[:, :, None], axis=1)
    hasP = jnp.sum(P, axis=1)
    sel = lax.dot_general(P, ic3, (((1,), (1,)), ((0,), (0,))),
                          preferred_element_type=jnp.float32)
    kio2 = lax.broadcasted_iota(jnp.int32, (TS, K), 1).astype(jnp.float32)
    t = kio2 - Cex_sel
    partial = jnp.sum(jnp.where(sel <= t[:, :, None], 1.0, 0.0), axis=2) * hasP
    idxi = (128.0 * F + partial).astype(jnp.int32)
    first = idxi[:, 0:1]
    idx_ref[...] = jnp.where(idxi >= N, jnp.broadcast_to(first, (TS, K)), idxi)


def bq_idx_pallas(xyz, cen, radius, K, interpret=False):
    B, N, _ = xyz.shape
    S = cen.shape[1]
    CH = N // 128
    r2 = np.float32(radius * radius)
    TS = min(128, S)
    cen_p = jnp.concatenate(
        [cen, jnp.zeros((B, S, 5), jnp.float32)], axis=-1)
    pts = xyz.transpose(0, 2, 1)
    idx = pl.pallas_call(
        functools.partial(_bqidx_kbody, N=N, K=K, CH=CH, r2=r2),
        grid=(B, S // TS),
        in_specs=[
            pl.BlockSpec((None, 3, N), lambda b, s: (b, 0, 0)),
            pl.BlockSpec((None, TS, 8), lambda b, s: (b, s, 0)),
        ],
        out_specs=pl.BlockSpec((None, TS, K), lambda b, s: (b, s, 0)),
        out_shape=jax.ShapeDtypeStruct((B, S, K), jnp.int32),
        interpret=interpret,
    )(pts, cen_p)
    return idx


def _mlp(x, layers):
    for (W, b) in layers:
        x = jax.nn.relu((x @ W + b) * _BN)
    return x


def _gath(x, idx):
    if idx.ndim == 2:
        return jnp.take_along_axis(x, idx[..., None], axis=1)
    B, S, K = idx.shape
    g = jnp.take_along_axis(x, idx.reshape(B, S * K)[..., None], axis=1)
    return g.reshape(B, S, K, x.shape[-1])


def _bq(radius, nsample, xyz, new_xyz):
    d2 = jnp.sum((new_xyz[:, :, None, :] - xyz[:, None, :, :]) ** 2, -1)
    N = xyz.shape[1]
    order = jnp.where(d2 <= radius * radius,
                      jnp.arange(N, dtype=jnp.float32)[None, None, :], jnp.inf)
    _, idx = jax.lax.top_k(-order, nsample)
    valid = jnp.take_along_axis(order, idx, axis=-1) < jnp.inf
    idx = jnp.where(valid, idx, idx[:, :, :1])
    return idx


def _sa_x(xyz, feats, npoint, radius, nsample, layers):
    B, N, _ = xyz.shape
    C = feats.shape[-1]
    K = nsample
    inds, new_xyz = fps2_pallas(xyz, npoint)
    idx = bq_idx_pallas(xyz, new_xyz, radius, K)
    Wpad = ((3 + C + 7) // 8) * 8
    table = jnp.concatenate(
        [xyz, feats, jnp.zeros((B, N, Wpad - 3 - C), jnp.float32)],
        axis=-1).reshape(B * N, Wpad)
    gidx = (idx + (jnp.arange(B, dtype=jnp.int32) * N)[:, None, None]).reshape(-1)
    rows = _gather_rows_sc(table, gidx)
    cen = jnp.concatenate(
        [new_xyz, jnp.zeros((B, npoint, 5), jnp.float32)], axis=-1
    ).reshape(B * npoint, 8)
    pooled = _mlp_max_pallas(rows, cen, layers, K, radius)
    return new_xyz, pooled.reshape(B, npoint, -1), inds


def _fp_kbody(u_ref, kT_ref, uf_ref, kf_ref, w1_ref, b1_ref, w2_ref, b2_ref,
              out_ref, *, NU, NK):
    ux = u_ref[...][:, 0:1]
    uy = u_ref[...][:, 1:2]
    uz = u_ref[...][:, 2:3]
    kx = kT_ref[0:1, :]
    ky = kT_ref[1:2, :]
    kz = kT_ref[2:3, :]
    dx = ux - kx
    dy = uy - ky
    dz = uz - kz
    d2 = dx * dx + dy * dy
    d2 = d2 + dz * dz
    iota_k = lax.broadcasted_iota(jnp.int32, (NU, NK), 1)
    A = jnp.zeros((NU, NK), jnp.float32)
    rsum = jnp.zeros((NU, 1), jnp.float32)
    for _ in range(3):
        mj = jnp.min(d2, axis=1, keepdims=True)
        idxj = jnp.min(jnp.where(d2 == mj, iota_k, NK), axis=1, keepdims=True)
        onehot = jnp.where(iota_k == idxj, 1.0, 0.0)
        recip = 1.0 / (mj + 1e-8)
        A = A + recip * onehot
        rsum = rsum + recip
        d2 = jnp.where(iota_k == idxj, jnp.inf, d2)
    A = A / rsum
    interp = jnp.dot(A, kf_ref[...], preferred_element_type=jnp.float32)
    x = jnp.concatenate([interp, uf_ref[...]], axis=1)
    h = jnp.dot(x, w1_ref[...], preferred_element_type=jnp.float32)
    h = jnp.maximum((h + b1_ref[...]) * _BN, 0.0)
    h = jnp.dot(h, w2_ref[...], preferred_element_type=jnp.float32)
    h = jnp.maximum((h + b2_ref[...]) * _BN, 0.0)
    out_ref[...] = h


def fp_pallas(unknown, known, unknown_feats, known_feats, layers, interpret=False):
    B, NU, _ = unknown.shape
    NK = known.shape[1]
    CU = unknown_feats.shape[-1]
    CK = known_feats.shape[-1]
    (W1, b1), (W2, b2) = layers
    Cout = W2.shape[1]
    kT = known.transpose(0, 2, 1)
    zer2 = lambda b: (0, 0)
    out = pl.pallas_call(
        functools.partial(_fp_kbody, NU=NU, NK=NK),
        grid=(B,),
        in_specs=[
            pl.BlockSpec((None, NU, 3), lambda b: (b, 0, 0)),
            pl.BlockSpec((None, 3, NK), lambda b: (b, 0, 0)),
            pl.BlockSpec((None, NU, CU), lambda b: (b, 0, 0)),
            pl.BlockSpec((None, NK, CK), lambda b: (b, 0, 0)),
            pl.BlockSpec(W1.shape, zer2), pl.BlockSpec((1, b1.shape[0]), zer2),
            pl.BlockSpec(W2.shape, zer2), pl.BlockSpec((1, b2.shape[0]), zer2),
        ],
        out_specs=pl.BlockSpec((None, NU, Cout), lambda b: (b, 0, 0)),
        out_shape=jax.ShapeDtypeStruct((B, NU, Cout), jnp.float32),
        interpret=interpret,
    )(unknown, kT, unknown_feats, known_feats, W1, b1[None], W2, b2[None])
    return out


def _fp_x(unknown, known, unknown_feats, known_feats, layers):
    return fp_pallas(unknown, known, unknown_feats, known_feats, layers)


def kernel(pointcloud, params):
    xyz = pointcloud[..., 0:3]
    feats = pointcloud[..., 3:]
    sa1_xyz, sa1_f, sa1_inds = _sa_x(xyz, feats, 2048, 0.2, 64, params['sa1'])
    sa2_xyz, sa2_f, sa2_inds = _sa_x(sa1_xyz, sa1_f, 1024, 0.4, 32, params['sa2'])
    sa3_xyz, sa3_f, sa3_inds = _sa_x(sa2_xyz, sa2_f, 512, 0.8, 16, params['sa3'])
    sa4_xyz, sa4_f, sa4_inds = _sa_x(sa3_xyz, sa3_f, 256, 1.2, 16, params['sa4'])
    f = _fp_x(sa3_xyz, sa4_xyz, sa3_f, sa4_f, params['fp1'])
    fp2_f = _fp_x(sa2_xyz, sa3_xyz, sa2_f, f, params['fp2'])
    fp2_inds = sa1_inds[:, :fp2_f.shape[1]]
    return (sa1_xyz, sa1_f, sa1_inds, sa2_xyz, sa2_f, sa2_inds, sa3_xyz, sa3_f,
            sa4_xyz, sa4_f, fp2_f, sa2_xyz, fp2_inds)
